# baseline (speedup 1.0000x reference)
import functools

import jax
import jax.numpy as jnp
from jax import lax
from jax.experimental import pallas as pl
from jax.experimental.pallas import tpu as pltpu
from jax.experimental.pallas import tpu_sc as plsc

_BN = 256
_BL = 256


def _tc_body(scal_ref, lp_ref, aa_ref, am_ref, mk_ref, out_ref,
             num_ref, den_ref, *, nv, nl):
    j = pl.program_id(1)

    @pl.when(j == 0)
    def _():
        num_ref[...] = jnp.zeros_like(num_ref)
        den_ref[...] = jnp.zeros_like(den_ref)

    x = lp_ref[...]
    aa = aa_ref[...]
    am = am_ref[...]
    mk = mk_ref[...]
    dead = mk == 0.0
    a = jnp.where(dead, 0, aa)
    b = jnp.where(dead, 0, am)
    acc0 = jnp.zeros(a.shape, jnp.float32)
    acc1 = jnp.zeros(a.shape, jnp.float32)
    for v in range(nv):
        xv = x[v]
        acc0 = acc0 + jnp.where(a == v, xv, 0.0)
        acc1 = acc1 + jnp.where(b == v, xv, 0.0)
    num_ref[...] += jnp.sum(acc0 - acc1, axis=-1)
    den_ref[...] += jnp.sum(mk, axis=-1)

    @pl.when(j == nl - 1)
    def _():
        out_ref[...] = scal_ref[0, 0] * num_ref[...] / den_ref[...]


def _seq_diffs(log_probs, aa, aa_mut, mask, boltzmann_scalar):
    n, l, v = log_probs.shape
    bn, bl = _BN, _BL
    lp_t = jnp.transpose(log_probs, (2, 0, 1))
    grid = (n // bn, l // bl)
    out = pl.pallas_call(
        functools.partial(_tc_body, nv=v, nl=l // bl),
        grid=grid,
        in_specs=[
            pl.BlockSpec(memory_space=pltpu.SMEM),
            pl.BlockSpec((v, bn, bl), lambda i, j: (0, i, j)),
            pl.BlockSpec((bn, bl), lambda i, j: (i, j)),
            pl.BlockSpec((bn, bl), lambda i, j: (i, j)),
            pl.BlockSpec((bn, bl), lambda i, j: (i, j)),
        ],
        out_specs=pl.BlockSpec((bn,), lambda i, j: (i,)),
        out_shape=jax.ShapeDtypeStruct((n,), jnp.float32),
        scratch_shapes=[
            pltpu.VMEM((bn,), jnp.float32),
            pltpu.VMEM((bn,), jnp.float32),
        ],
        compiler_params=pltpu.CompilerParams(
            dimension_semantics=("parallel", "arbitrary")),
    )(boltzmann_scalar.reshape(1, 1), lp_t, aa, aa_mut, mask)
    return out


def _make_sc_segment(n, g):
    mesh = plsc.VectorSubcoreMesh(core_axis_name="c", subcore_axis_name="s")
    f32 = jnp.float32

    @functools.partial(
        pl.kernel,
        mesh=mesh,
        out_type=(
            jax.ShapeDtypeStruct((g,), f32),
            jax.ShapeDtypeStruct((g,), f32),
            jax.ShapeDtypeStruct((16,), f32),
        ),
        scratch_types=[
            pltpu.VMEM((n,), f32),
            pltpu.VMEM((n + 16,), f32),
            pltpu.VMEM((n,), f32),
            pltpu.VMEM((g,), jnp.int32),
            pltpu.VMEM((g,), f32),
            pltpu.VMEM((g,), f32),
            pltpu.VMEM((16,), f32),
            pltpu.VMEM((16,), f32),
            pltpu.VMEM((16,), jnp.int32),
        ],
        compiler_params=pltpu.CompilerParams(needs_layout_passes=False),
    )
    def sc_segment(d_hbm, ddg_hbm, nmc_hbm, pred_hbm, tru_hbm, loss_hbm,
                   d_v, pfx_v, g_v, nmc_v, pred_v, tru_v, loss_v,
                   tf_v, ti_v):
        wid = lax.axis_index("s") * 2 + lax.axis_index("c")
        iota = lax.iota(jnp.int32, 16)
        last = jnp.full((16,), 15, jnp.int32)

        def scan16(x, tmp):
            del tmp
            for s in (1, 2, 4, 8):
                sh = x.at[jnp.maximum(iota - s, 0)].get(
                    mode="promise_in_bounds")
                x = x + jnp.where(iota >= s, sh, jnp.zeros_like(x))
            return x

        def bcast_last(x, tmp):
            del tmp
            return x.at[last].get(mode="promise_in_bounds")

        @pl.when(wid == 0)
        def _():
            pltpu.sync_copy(d_hbm, d_v)
            pltpu.sync_copy(ddg_hbm, g_v)
            pltpu.sync_copy(nmc_hbm, nmc_v)

            def prefix_body(i, carry):
                b = pl.multiple_of(i * 16, 8)
                v = d_v[pl.ds(b, 16)]
                incl = scan16(v, tf_v)
                pfx_v[pl.ds(b, 16)] = (incl - v) + carry
                return carry + bcast_last(incl, tf_v)

            tot = lax.fori_loop(0, n // 16, prefix_body,
                                jnp.zeros((16,), f32))
            pfx_v[pl.ds(n, 16)] = tot

            def seg_body(i, carry):
                off, acc = carry
                b = pl.multiple_of(i * 16, 8)
                v = nmc_v[pl.ds(b, 16)]
                incl = scan16(v, ti_v)
                c = iota + b + off + (incl - v)
                nxt = c + v + 1
                d_c = plsc.load_gather(d_v, [c])
                p_c = plsc.load_gather(pfx_v, [c])
                p_n = plsc.load_gather(pfx_v, [nxt])
                tru = plsc.load_gather(g_v, [c])
                pred = 2.0 * d_c - (p_n - p_c)
                pred_v[pl.ds(b, 16)] = pred
                tru_v[pl.ds(b, 16)] = tru
                e = pred - tru
                return off + bcast_last(incl, ti_v), acc + e * e

            _, acc = lax.fori_loop(
                0, g // 16, seg_body,
                (jnp.zeros((16,), jnp.int32), jnp.zeros((16,), f32)))
            tot_l = bcast_last(scan16(acc, tf_v), tf_v)
            loss_v[...] = tot_l * (1.0 / g)
            pltpu.sync_copy(pred_v, pred_hbm)
            pltpu.sync_copy(tru_v, tru_hbm)
            pltpu.sync_copy(loss_v, loss_hbm)

    return sc_segment


def kernel(log_probs, aa, aa_mut, mask, num_mut_chains, ddG, boltzmann_scalar):
    n = log_probs.shape[0]
    g = num_mut_chains.shape[0]
    d = _seq_diffs(log_probs, aa, aa_mut, mask, boltzmann_scalar)
    pred, tru, loss16 = _make_sc_segment(n, g)(d, ddG, num_mut_chains)
    return (loss16[0], pred, tru)


# speedup vs baseline: 1.1069x; 1.1069x over previous
import functools

import jax
import jax.numpy as jnp
from jax import lax
from jax.experimental import pallas as pl
from jax.experimental.pallas import tpu as pltpu
from jax.experimental.pallas import tpu_sc as plsc

_BN = 256
_BL = 512


def _tc_body(scal_ref, lp_ref, aa_ref, am_ref, mk_ref, out_ref, *, nv):
    x = lp_ref[...]
    aa = aa_ref[...]
    am = am_ref[...]
    mk = mk_ref[...]
    dead = mk == 0.0
    a = jnp.where(dead, 0, aa)
    b = jnp.where(dead, 0, am)
    acc0 = jnp.zeros(a.shape, jnp.float32)
    acc1 = jnp.zeros(a.shape, jnp.float32)
    for v in range(nv):
        xv = x[v]
        acc0 = acc0 + jnp.where(a == v, xv, 0.0)
        acc1 = acc1 + jnp.where(b == v, xv, 0.0)
    num = jnp.sum(acc0 - acc1, axis=-1)
    den = jnp.sum(mk, axis=-1)
    out_ref[...] = scal_ref[0, 0] * num / den


def _seq_diffs(log_probs, aa, aa_mut, mask, boltzmann_scalar):
    n, l, v = log_probs.shape
    bn, bl = _BN, _BL
    lp_t = jnp.transpose(log_probs, (2, 0, 1))
    grid = (n // bn,)
    out = pl.pallas_call(
        functools.partial(_tc_body, nv=v),
        grid=grid,
        in_specs=[
            pl.BlockSpec(memory_space=pltpu.SMEM),
            pl.BlockSpec((v, bn, bl), lambda i: (0, i, 0)),
            pl.BlockSpec((bn, bl), lambda i: (i, 0)),
            pl.BlockSpec((bn, bl), lambda i: (i, 0)),
            pl.BlockSpec((bn, bl), lambda i: (i, 0)),
        ],
        out_specs=pl.BlockSpec((bn,), lambda i: (i,)),
        out_shape=jax.ShapeDtypeStruct((n,), jnp.float32),
        compiler_params=pltpu.CompilerParams(
            dimension_semantics=("parallel",)),
    )(boltzmann_scalar.reshape(1, 1), lp_t, aa, aa_mut, mask)
    return out


def _make_sc_segment(n, g):
    mesh = plsc.VectorSubcoreMesh(core_axis_name="c", subcore_axis_name="s")
    f32 = jnp.float32

    @functools.partial(
        pl.kernel,
        mesh=mesh,
        out_type=(
            jax.ShapeDtypeStruct((g,), f32),
            jax.ShapeDtypeStruct((g,), f32),
            jax.ShapeDtypeStruct((16,), f32),
        ),
        scratch_types=[
            pltpu.VMEM((n,), f32),
            pltpu.VMEM((n + 16,), f32),
            pltpu.VMEM((n,), f32),
            pltpu.VMEM((g,), jnp.int32),
            pltpu.VMEM((g,), f32),
            pltpu.VMEM((g,), f32),
            pltpu.VMEM((16,), f32),
            pltpu.VMEM((16,), f32),
            pltpu.VMEM((16,), jnp.int32),
        ],
        compiler_params=pltpu.CompilerParams(needs_layout_passes=False),
    )
    def sc_segment(d_hbm, ddg_hbm, nmc_hbm, pred_hbm, tru_hbm, loss_hbm,
                   d_v, pfx_v, g_v, nmc_v, pred_v, tru_v, loss_v,
                   tf_v, ti_v):
        wid = lax.axis_index("s") * 2 + lax.axis_index("c")
        iota = lax.iota(jnp.int32, 16)
        last = jnp.full((16,), 15, jnp.int32)

        def scan16(x, tmp):
            del tmp
            for s in (1, 2, 4, 8):
                sh = x.at[jnp.maximum(iota - s, 0)].get(
                    mode="promise_in_bounds")
                x = x + jnp.where(iota >= s, sh, jnp.zeros_like(x))
            return x

        def bcast_last(x, tmp):
            del tmp
            return x.at[last].get(mode="promise_in_bounds")

        @pl.when(wid == 0)
        def _():
            pltpu.sync_copy(d_hbm, d_v)
            pltpu.sync_copy(ddg_hbm, g_v)
            pltpu.sync_copy(nmc_hbm, nmc_v)

            def prefix_body(i, carry):
                b = pl.multiple_of(i * 16, 8)
                v = d_v[pl.ds(b, 16)]
                incl = scan16(v, tf_v)
                pfx_v[pl.ds(b, 16)] = (incl - v) + carry
                return carry + bcast_last(incl, tf_v)

            tot = lax.fori_loop(0, n // 16, prefix_body,
                                jnp.zeros((16,), f32))
            pfx_v[pl.ds(n, 16)] = tot

            def seg_body(i, carry):
                off, acc = carry
                b = pl.multiple_of(i * 16, 8)
                v = nmc_v[pl.ds(b, 16)]
                incl = scan16(v, ti_v)
                c = iota + b + off + (incl - v)
                nxt = c + v + 1
                d_c = plsc.load_gather(d_v, [c])
                p_c = plsc.load_gather(pfx_v, [c])
                p_n = plsc.load_gather(pfx_v, [nxt])
                tru = plsc.load_gather(g_v, [c])
                pred = 2.0 * d_c - (p_n - p_c)
                pred_v[pl.ds(b, 16)] = pred
                tru_v[pl.ds(b, 16)] = tru
                e = pred - tru
                return off + bcast_last(incl, ti_v), acc + e * e

            _, acc = lax.fori_loop(
                0, g // 16, seg_body,
                (jnp.zeros((16,), jnp.int32), jnp.zeros((16,), f32)))
            tot_l = bcast_last(scan16(acc, tf_v), tf_v)
            loss_v[...] = tot_l * (1.0 / g)
            pltpu.sync_copy(pred_v, pred_hbm)
            pltpu.sync_copy(tru_v, tru_hbm)
            pltpu.sync_copy(loss_v, loss_hbm)

    return sc_segment


def kernel(log_probs, aa, aa_mut, mask, num_mut_chains, ddG, boltzmann_scalar):
    n = log_probs.shape[0]
    g = num_mut_chains.shape[0]
    d = _seq_diffs(log_probs, aa, aa_mut, mask, boltzmann_scalar)
    pred, tru, loss16 = _make_sc_segment(n, g)(d, ddG, num_mut_chains)
    return (loss16[0], pred, tru)
